# baseline (speedup 1.0000x reference)
import functools

import jax
import jax.numpy as jnp
import numpy as np
from jax import lax
from jax.experimental import pallas as pl
from jax.experimental.pallas import tpu as pltpu
from jax.experimental.pallas import tpu_sc as plsc

B, L = 4096, 200
D = 64
HIDDEN = 256
NUM_CLASSES = 10

NC, NS = 2, 16
NW = NC * NS
BPW = B // NW
IPW = BPW * L
CH0, CH1 = 104, 96
HALF = BPW // 2

_mesh = plsc.VectorSubcoreMesh(core_axis_name="c", subcore_axis_name="s")


@functools.partial(
    pl.kernel,
    out_type=jax.ShapeDtypeStruct((B, 2 * D), jnp.float32),
    mesh=_mesh,
    scratch_types=[
        pltpu.VMEM((IPW,), jnp.int32),
        pltpu.VMEM((IPW + 16,), jnp.int32),
        pltpu.VMEM((2, L, 128), jnp.bfloat16),
        pltpu.VMEM((BPW, 2 * D), jnp.float32),
        pltpu.SemaphoreType.DMA,
        pltpu.SemaphoreType.DMA,
    ],
    compiler_params=pltpu.CompilerParams(
        use_tc_tiling_on_sc=False, needs_layout_passes=False),
)
def _pool(wh_hbm, wp_hbm, bh_hbm, bp_hbm, emb_hbm, embb_hbm, out_hbm,
          idx_v, par_v, buf_v, out_v, sem0, sem1):
    wid = lax.axis_index("c") * NS + lax.axis_index("s")
    ibase = wid * IPW

    himask = jnp.full((16,), 0xFFFF0000, jnp.uint32).astype(jnp.int32)
    inv_l = jnp.float32(1.0 / L)

    def phase(table_hbm, h_hbm, p_hbm, col):
        pltpu.sync_copy(h_hbm.at[pl.ds(ibase, IPW)], idx_v)
        pltpu.sync_copy(p_hbm.at[pl.ds(ibase, IPW)], par_v.at[pl.ds(0, IPW)])

        def issue(r, slot, sem):
            pltpu.async_copy(
                table_hbm.at[idx_v.at[pl.ds(r * L, CH0)]],
                buf_v.at[slot, pl.ds(0, CH0)], sem)
            pltpu.async_copy(
                table_hbm.at[idx_v.at[pl.ds(r * L + CH0, CH1)]],
                buf_v.at[slot, pl.ds(CH0, CH1)], sem)

        def drain(r, slot, sem):
            pltpu.make_async_copy(
                table_hbm.at[idx_v.at[pl.ds(r * L, CH0)]],
                buf_v.at[slot, pl.ds(0, CH0)], sem).wait()
            pltpu.make_async_copy(
                table_hbm.at[idx_v.at[pl.ds(r * L + CH0, CH1)]],
                buf_v.at[slot, pl.ds(CH0, CH1)], sem).wait()

        def reduce(r, slot):
            def rbody(g, accs):
                a0, a1, a2, a3 = accs
                pvec = par_v[pl.ds(r * L + g * 8, 16)]
                for k in range(8):
                    take_hi = pvec[k] != 0
                    row = buf_v.at[slot, g * 8 + k]
                    lo0 = plsc.bitcast(row[pl.ds(0, 32)], jnp.int32)
                    lo1 = plsc.bitcast(row[pl.ds(32, 32)], jnp.int32)
                    hi0 = plsc.bitcast(row[pl.ds(64, 32)], jnp.int32)
                    hi1 = plsc.bitcast(row[pl.ds(96, 32)], jnp.int32)
                    s0 = jnp.where(take_hi, hi0, lo0)
                    s1 = jnp.where(take_hi, hi1, lo1)
                    a0 = a0 + plsc.bitcast(s0 << 16, jnp.float32)
                    a1 = a1 + plsc.bitcast(s0 & himask, jnp.float32)
                    a2 = a2 + plsc.bitcast(s1 << 16, jnp.float32)
                    a3 = a3 + plsc.bitcast(s1 & himask, jnp.float32)
                return a0, a1, a2, a3

            z = jnp.zeros((16,), jnp.float32)
            accs = lax.fori_loop(0, L // 8, rbody, (z, z, z, z))
            for d in range(4):
                out_v[r, pl.ds(col + 16 * d, 16)] = accs[d] * inv_l

        issue(0, 0, sem0)
        issue(1, 1, sem1)

        def body(r2, carry):
            r0 = 2 * r2
            drain(r0, 0, sem0)
            reduce(r0, 0)

            @pl.when(r2 < HALF - 1)
            def _():
                issue(r0 + 2, 0, sem0)

            drain(r0 + 1, 1, sem1)
            reduce(r0 + 1, 1)

            @pl.when(r2 < HALF - 1)
            def _():
                issue(r0 + 3, 1, sem1)

            return carry

        lax.fori_loop(0, HALF, body, 0)

    phase(emb_hbm, wh_hbm, wp_hbm, 0)
    phase(embb_hbm, bh_hbm, bp_hbm, D)

    pltpu.sync_copy(out_v, out_hbm.at[pl.ds(wid * BPW, BPW)])


def _mlp_body(x_ref, w1_ref, b1_ref, w2_ref, b2_ref, o_ref):
    h = jnp.dot(x_ref[...], w1_ref[...], preferred_element_type=jnp.float32)
    h = jnp.maximum(h + b1_ref[...], 0.0)
    o = jnp.dot(h, w2_ref[...], preferred_element_type=jnp.float32)
    o_ref[...] = o + b2_ref[...]


_BM = 512


def _mlp(pooled, w1t, b1r, w2p, b2p):
    return pl.pallas_call(
        _mlp_body,
        grid=(B // _BM,),
        in_specs=[
            pl.BlockSpec((_BM, 2 * D), lambda i: (i, 0)),
            pl.BlockSpec((2 * D, HIDDEN), lambda i: (0, 0)),
            pl.BlockSpec((1, HIDDEN), lambda i: (0, 0)),
            pl.BlockSpec((HIDDEN, 128), lambda i: (0, 0)),
            pl.BlockSpec((1, 128), lambda i: (0, 0)),
        ],
        out_specs=pl.BlockSpec((_BM, 128), lambda i: (i, 0)),
        out_shape=jax.ShapeDtypeStruct((B, 128), jnp.float32),
    )(pooled, w1t, b1r, w2p, b2p)


def _pooled_perm():
    blk = np.concatenate([
        np.arange(0, 32, 2), np.arange(1, 32, 2),
        np.arange(32, 64, 2), np.arange(33, 64, 2),
    ])
    return np.concatenate([blk, blk + 64])


_PERM = _pooled_perm()


def kernel(words, bigram, emb, emb_bigram, W1, b1, W2, b2):
    wflat = words.reshape(-1)
    bflat = bigram.reshape(-1)
    wh, wp = wflat >> 1, wflat & 1
    bh, bp = bflat >> 1, bflat & 1
    emb16 = emb.reshape(-1, 2 * D).astype(jnp.bfloat16)
    embb16 = emb_bigram.reshape(-1, 2 * D).astype(jnp.bfloat16)

    pooled = _pool(wh, wp, bh, bp, emb16, embb16)

    w1t = W1.T[_PERM, :]
    b1r = b1.reshape(1, HIDDEN)
    w2p = jnp.zeros((HIDDEN, 128), W2.dtype).at[:, :NUM_CLASSES].set(W2.T)
    b2p = jnp.zeros((1, 128), b2.dtype).at[0, :NUM_CLASSES].set(b2)
    out = _mlp(pooled, w1t, b1r, w2p, b2p)
    return out[:, :NUM_CLASSES]
